# Initial kernel scaffold; baseline (speedup 1.0000x reference)
import jax
import jax.numpy as jnp
from jax.experimental import pallas as pl


def kernel(x, edgeIndex, edgeWeight, W_self, b_self, W_neigh, b_neigh, gamma, beta):
    raise NotImplementedError("write your pallas kernel here")


# speedup vs baseline: 4.1712x; 4.1712x over previous
import functools

import jax
import jax.numpy as jnp
from jax import lax
from jax.experimental import pallas as pl
from jax.experimental.pallas import tpu as pltpu
from jax.experimental.pallas import tpu_sc as plsc

NC = 2
NS = 16
NW = NC * NS
CH = 128


def _prep_body(x_ref, wn_ref, ws_ref, bias_ref, y_ref, z_ref):
    xb = x_ref[...]
    dn = (((1,), (1,)), ((), ()))
    y_ref[...] = lax.dot_general(xb, wn_ref[...], dn,
                                 preferred_element_type=jnp.float32)
    z_ref[...] = lax.dot_general(xb, ws_ref[...], dn,
                                 preferred_element_type=jnp.float32) + bias_ref[...]


def _prep(x, W_neigh, W_self, b_self, b_neigh):
    N, D = x.shape
    B = 2000 if N % 2000 == 0 else 8
    grid = (N // B,)
    bias = (b_self + b_neigh).reshape(1, D)
    return pl.pallas_call(
        _prep_body,
        grid=grid,
        in_specs=[
            pl.BlockSpec((B, D), lambda i: (i, 0)),
            pl.BlockSpec((D, D), lambda i: (0, 0)),
            pl.BlockSpec((D, D), lambda i: (0, 0)),
            pl.BlockSpec((1, D), lambda i: (0, 0)),
        ],
        out_specs=[
            pl.BlockSpec((B, D), lambda i: (i, 0)),
            pl.BlockSpec((B, D), lambda i: (i, 0)),
        ],
        out_shape=[
            jax.ShapeDtypeStruct((N, D), jnp.float32),
            jax.ShapeDtypeStruct((N, D), jnp.float32),
        ],
    )(x, W_neigh, W_self, bias)


def _scatter_sc(y, src, dst, w):
    N, D = y.shape
    E = src.shape[0]
    E_pad = ((E + NW * CH - 1) // (NW * CH)) * (NW * CH)
    pad = E_pad - E
    if pad:
        src = jnp.concatenate([src, jnp.zeros((pad,), jnp.int32)])
        dst = jnp.concatenate([dst, jnp.zeros((pad,), jnp.int32)])
        w = jnp.concatenate([w, jnp.zeros((pad,), jnp.float32)])
    n_ch = E_pad // (NW * CH)
    src = src.reshape(NW, n_ch, CH)
    dst = dst.reshape(NW, n_ch, CH)
    w = w.reshape(NW, n_ch, CH)
    rows_per_tile = ((N + NS * 8 - 1) // (NS * 8)) * 8
    N_pad = rows_per_tile * NS
    zeros_init = jnp.zeros((N_pad, D), jnp.float32)
    nvec = D // 16

    mesh = plsc.VectorSubcoreMesh(core_axis_name="c", subcore_axis_name="s")

    @functools.partial(
        pl.kernel,
        out_type=[
            jax.ShapeDtypeStruct((N_pad, D), jnp.float32),
            jax.ShapeDtypeStruct((N_pad, D), jnp.float32),
        ],
        mesh=mesh,
        scratch_types=[
            pltpu.VMEM_SHARED((N_pad, D), jnp.float32),
            pltpu.VMEM((n_ch, CH), jnp.int32),
            pltpu.VMEM((n_ch, CH), jnp.int32),
            pltpu.VMEM((n_ch, CH), jnp.float32),
            pltpu.VMEM((CH, D), jnp.float32),
            pltpu.SemaphoreType.DMA,
        ],
    )
    def sc_kernel(y_hbm, src_hbm, dst_hbm, w_hbm, zero_hbm, p0_hbm, p1_hbm,
                  acc, src_v, dst_v, w_v, rows, sem):
        c = lax.axis_index("c")
        s = lax.axis_index("s")
        wid = s * NC + c
        tile_rows = pl.ds(s * rows_per_tile, rows_per_tile)

        pltpu.sync_copy(zero_hbm.at[tile_rows], acc.at[tile_rows])
        pltpu.sync_copy(src_hbm.at[wid], src_v)
        pltpu.sync_copy(dst_hbm.at[wid], dst_v)
        pltpu.sync_copy(w_hbm.at[wid], w_v)
        plsc.subcore_barrier()

        def chunk_body(b, carry):
            pltpu.async_copy(y_hbm.at[src_v.at[b]], rows, sem).wait()
            for g in range(CH // 16):
                wv = w_v[b, pl.ds(g * 16, 16)]
                for t in range(16):
                    e = g * 16 + t
                    w_e = wv[t]
                    for j in range(nvec):
                        sl = pl.ds(j * 16, 16)
                        rows[e, sl] = rows[e, sl] * w_e
            pltpu.sync_copy(rows, acc.at[dst_v.at[b]], add=True)
            return carry

        lax.fori_loop(0, n_ch, chunk_body, 0)
        plsc.subcore_barrier()

        @pl.when(c == 0)
        def _():
            pltpu.sync_copy(acc.at[tile_rows], p0_hbm.at[tile_rows])

        @pl.when(c == 1)
        def _():
            pltpu.sync_copy(acc.at[tile_rows], p1_hbm.at[tile_rows])

    return sc_kernel(y, src, dst, w, zeros_init)


def _post_body(z_ref, p0_ref, p1_ref, g_ref, b_ref, o_ref):
    h = z_ref[...] + p0_ref[...] + p1_ref[...]
    h = jnp.maximum(h, 0.0)
    mean = jnp.mean(h, axis=1, keepdims=True)
    cen = h - mean
    var = jnp.mean(cen * cen, axis=1, keepdims=True)
    o_ref[...] = cen * lax.rsqrt(var + 1e-5) * g_ref[...] + b_ref[...]


def _post(z, p0, p1, gamma, beta):
    N, D = z.shape
    B = 2000 if N % 2000 == 0 else 8
    grid = (N // B,)
    return pl.pallas_call(
        _post_body,
        grid=grid,
        in_specs=[
            pl.BlockSpec((B, D), lambda i: (i, 0)),
            pl.BlockSpec((B, D), lambda i: (i, 0)),
            pl.BlockSpec((B, D), lambda i: (i, 0)),
            pl.BlockSpec((1, D), lambda i: (0, 0)),
            pl.BlockSpec((1, D), lambda i: (0, 0)),
        ],
        out_specs=pl.BlockSpec((B, D), lambda i: (i, 0)),
        out_shape=jax.ShapeDtypeStruct((N, D), jnp.float32),
    )(z, p0, p1, gamma.reshape(1, D), beta.reshape(1, D))


def kernel(x, edgeIndex, edgeWeight, W_self, b_self, W_neigh, b_neigh, gamma, beta):
    y, z = _prep(x, W_neigh, W_self, b_self, b_neigh)
    p0, p1 = _scatter_sc(y, edgeIndex[0], edgeIndex[1], edgeWeight)
    return _post(z, p0, p1, gamma, beta)


# speedup vs baseline: 6.2627x; 1.5014x over previous
import functools

import jax
import jax.numpy as jnp
from jax import lax
from jax.experimental import pallas as pl
from jax.experimental.pallas import tpu as pltpu
from jax.experimental.pallas import tpu_sc as plsc

NC = 2
NS = 16
NW = NC * NS
CH = 80


def _prep_body(x_ref, wn_ref, ws_ref, bias_ref, y_ref, z_ref):
    xb = x_ref[...]
    dn = (((1,), (1,)), ((), ()))
    y_ref[...] = lax.dot_general(xb, wn_ref[...], dn,
                                 preferred_element_type=jnp.float32)
    z_ref[...] = lax.dot_general(xb, ws_ref[...], dn,
                                 preferred_element_type=jnp.float32) + bias_ref[...]


def _prep(x, W_neigh, W_self, b_self, b_neigh):
    N, D = x.shape
    B = 2000 if N % 2000 == 0 else 8
    grid = (N // B,)
    bias = (b_self + b_neigh).reshape(1, D)
    return pl.pallas_call(
        _prep_body,
        grid=grid,
        in_specs=[
            pl.BlockSpec((B, D), lambda i: (i, 0)),
            pl.BlockSpec((D, D), lambda i: (0, 0)),
            pl.BlockSpec((D, D), lambda i: (0, 0)),
            pl.BlockSpec((1, D), lambda i: (0, 0)),
        ],
        out_specs=[
            pl.BlockSpec((B, D), lambda i: (i, 0)),
            pl.BlockSpec((B, D), lambda i: (i, 0)),
        ],
        out_shape=[
            jax.ShapeDtypeStruct((N, D), jnp.float32),
            jax.ShapeDtypeStruct((N, D), jnp.float32),
        ],
    )(x, W_neigh, W_self, bias)


def _scatter_sc(y, src, dst, w):
    N, D = y.shape
    E = src.shape[0]
    E_pad = ((E + 2 * NW * CH - 1) // (2 * NW * CH)) * (2 * NW * CH)
    pad = E_pad - E
    if pad:
        src = jnp.concatenate([src, jnp.zeros((pad,), jnp.int32)])
        dst = jnp.concatenate([dst, jnp.zeros((pad,), jnp.int32)])
        w = jnp.concatenate([w, jnp.zeros((pad,), jnp.float32)])
    n_ch = E_pad // (NW * CH)
    src = src.reshape(NW, n_ch, CH)
    dst = dst.reshape(NW, n_ch, CH)
    w = w.reshape(NW, n_ch, CH)
    rows_per_tile = ((N + NS * 8 - 1) // (NS * 8)) * 8
    N_pad = rows_per_tile * NS
    zeros_init = jnp.zeros((N_pad, D), jnp.float32)
    nvec = D // 16

    mesh = plsc.VectorSubcoreMesh(core_axis_name="c", subcore_axis_name="s")

    @functools.partial(
        pl.kernel,
        out_type=[
            jax.ShapeDtypeStruct((N_pad, D), jnp.float32),
            jax.ShapeDtypeStruct((N_pad, D), jnp.float32),
        ],
        mesh=mesh,
        scratch_types=[
            pltpu.VMEM_SHARED((N_pad, D), jnp.float32),
            pltpu.VMEM((4, CH), jnp.int32),
            pltpu.VMEM((4, CH), jnp.int32),
            pltpu.VMEM((2, CH), jnp.float32),
            pltpu.VMEM((2, CH, D), jnp.float32),
            pltpu.VMEM((2, CH, D), jnp.float32),
            pltpu.SemaphoreType.DMA,
            pltpu.SemaphoreType.DMA,
            pltpu.SemaphoreType.DMA,
            pltpu.SemaphoreType.DMA,
            pltpu.SemaphoreType.DMA,
            pltpu.SemaphoreType.DMA,
            pltpu.SemaphoreType.DMA,
        ],
    )
    def sc_kernel(y_hbm, src_hbm, dst_hbm, w_hbm, zero_hbm, p0_hbm, p1_hbm,
                  acc, src_r, dst_r, w_r, rows_in, rows_out,
                  gsem0, gsem1, ssem0, ssem1, srcsem, dstsem, wsem):
        c = lax.axis_index("c")
        s = lax.axis_index("s")
        wid = s * NC + c
        tile_rows = pl.ds(s * rows_per_tile, rows_per_tile)
        gsem = (gsem0, gsem1)
        ssem = (ssem0, ssem1)

        def load_src(b, ring_slot):
            pltpu.make_async_copy(
                src_hbm.at[wid, b], src_r.at[ring_slot], srcsem).start()

        def load_dst(b, ring_slot):
            pltpu.make_async_copy(
                dst_hbm.at[wid, b], dst_r.at[ring_slot], dstsem).start()

        def load_w(b, slot):
            pltpu.make_async_copy(
                w_hbm.at[wid, b], w_r.at[slot], wsem).start()

        pltpu.sync_copy(zero_hbm.at[tile_rows], acc.at[tile_rows])

        for b in range(4):
            load_src(b, b)
            load_dst(b, b)
        for slot in range(2):
            load_w(slot, slot)
        plsc.subcore_barrier()

        for slot in range(2):
            pltpu.make_async_copy(
                src_hbm.at[wid, slot], src_r.at[slot], srcsem).wait()
            pltpu.make_async_copy(
                y_hbm.at[src_r.at[slot]], rows_in.at[slot], gsem[slot]).start()

        n2 = n_ch // 2

        def chunk_pair(b2, carry):
            r2 = lax.rem(b2, 2)
            for slot in range(2):
                b = 2 * b2 + slot
                i4 = 2 * r2 + slot
                i4n = 2 * (1 - r2) + slot
                rin = rows_in.at[slot]
                rout = rows_out.at[slot]
                pltpu.make_async_copy(
                    y_hbm.at[src_r.at[i4]], rin, gsem[slot]
                ).wait()

                @pl.when(b2 >= 1)
                def _():
                    pltpu.make_async_copy(
                        rout, acc.at[dst_r.at[i4n]], ssem[slot]
                    ).wait()

                @pl.when(jnp.logical_and(b2 >= 1, b + 2 < n_ch))
                def _():
                    load_dst(b + 2, i4n)

                pltpu.make_async_copy(
                    w_hbm.at[wid, b], w_r.at[slot], wsem
                ).wait()

                for g in range(CH // 16):
                    wv = w_r[slot, pl.ds(g * 16, 16)]
                    for t in range(16):
                        e = g * 16 + t
                        w_e = wv[t]
                        for j in range(nvec):
                            sl = pl.ds(j * 16, 16)
                            rout[e, sl] = rin[e, sl] * w_e

                @pl.when(b + 2 < n_ch)
                def _():
                    load_w(b + 2, slot)

                @pl.when(b + 4 < n_ch)
                def _():
                    load_src(b + 4, i4)

                @pl.when(b + 2 < n_ch)
                def _():
                    pltpu.make_async_copy(
                        src_hbm.at[wid, b + 2], src_r.at[i4n], srcsem).wait()
                    pltpu.make_async_copy(
                        y_hbm.at[src_r.at[i4n]], rin, gsem[slot]).start()

                pltpu.make_async_copy(
                    dst_hbm.at[wid, b], dst_r.at[i4], dstsem).wait()
                pltpu.make_async_copy(
                    rout, acc.at[dst_r.at[i4]], ssem[slot]
                ).start(add=True)
            return carry

        lax.fori_loop(0, n2, chunk_pair, 0)
        for slot in range(2):
            pltpu.make_async_copy(
                rows_out.at[slot], acc.at[dst_r.at[slot]], ssem[slot]
            ).wait()
        plsc.subcore_barrier()

        @pl.when(c == 0)
        def _():
            pltpu.sync_copy(acc.at[tile_rows], p0_hbm.at[tile_rows])

        @pl.when(c == 1)
        def _():
            pltpu.sync_copy(acc.at[tile_rows], p1_hbm.at[tile_rows])

    return sc_kernel(y, src, dst, w, zeros_init)


def _post_body(z_ref, p0_ref, p1_ref, g_ref, b_ref, o_ref):
    h = z_ref[...] + p0_ref[...] + p1_ref[...]
    h = jnp.maximum(h, 0.0)
    mean = jnp.mean(h, axis=1, keepdims=True)
    cen = h - mean
    var = jnp.mean(cen * cen, axis=1, keepdims=True)
    o_ref[...] = cen * lax.rsqrt(var + 1e-5) * g_ref[...] + b_ref[...]


def _post(z, p0, p1, gamma, beta):
    N, D = z.shape
    B = 2000 if N % 2000 == 0 else 8
    grid = (N // B,)
    return pl.pallas_call(
        _post_body,
        grid=grid,
        in_specs=[
            pl.BlockSpec((B, D), lambda i: (i, 0)),
            pl.BlockSpec((B, D), lambda i: (i, 0)),
            pl.BlockSpec((B, D), lambda i: (i, 0)),
            pl.BlockSpec((1, D), lambda i: (0, 0)),
            pl.BlockSpec((1, D), lambda i: (0, 0)),
        ],
        out_specs=pl.BlockSpec((B, D), lambda i: (i, 0)),
        out_shape=jax.ShapeDtypeStruct((N, D), jnp.float32),
    )(z, p0, p1, gamma.reshape(1, D), beta.reshape(1, D))


def kernel(x, edgeIndex, edgeWeight, W_self, b_self, W_neigh, b_neigh, gamma, beta):
    y, z = _prep(x, W_neigh, W_self, b_self, b_neigh)
    p0, p1 = _scatter_sc(y, edgeIndex[0], edgeIndex[1], edgeWeight)
    return _post(z, p0, p1, gamma, beta)


# speedup vs baseline: 9.0483x; 1.4448x over previous
import functools

import jax
import jax.numpy as jnp
from jax import lax
from jax.experimental import pallas as pl
from jax.experimental.pallas import tpu as pltpu
from jax.experimental.pallas import tpu_sc as plsc

NC = 2
NS = 16
NW = NC * NS
CH = 80


def _prep_body(x_ref, wn_ref, ws_ref, bias_ref, y_ref, z_ref):
    xb = x_ref[...]
    dn = (((1,), (1,)), ((), ()))
    y_ref[...] = lax.dot_general(xb, wn_ref[...], dn,
                                 preferred_element_type=jnp.float32)
    z_ref[...] = lax.dot_general(xb, ws_ref[...], dn,
                                 preferred_element_type=jnp.float32) + bias_ref[...]


def _prep(x, W_neigh, W_self, b_self, b_neigh):
    N, D = x.shape
    B = 2000 if N % 2000 == 0 else 8
    grid = (N // B,)
    bias = (b_self + b_neigh).reshape(1, D)
    return pl.pallas_call(
        _prep_body,
        grid=grid,
        in_specs=[
            pl.BlockSpec((B, D), lambda i: (i, 0)),
            pl.BlockSpec((D, D), lambda i: (0, 0)),
            pl.BlockSpec((D, D), lambda i: (0, 0)),
            pl.BlockSpec((1, D), lambda i: (0, 0)),
        ],
        out_specs=[
            pl.BlockSpec((B, D), lambda i: (i, 0)),
            pl.BlockSpec((B, D), lambda i: (i, 0)),
        ],
        out_shape=[
            jax.ShapeDtypeStruct((N, D), jnp.float32),
            jax.ShapeDtypeStruct((N, D), jnp.float32),
        ],
    )(x, W_neigh, W_self, bias)


FRAC0 = 0.62


def _scatter_sc(y, src, dst, w):
    N, D = y.shape
    E = src.shape[0]
    n_pt = ((E + 2 * NS * CH - 1) // (2 * NS * CH)) * 2
    n0 = max(4, int(round(FRAC0 * n_pt / 2)) * 2)
    n1 = n_pt - n0
    assert n1 >= 4
    E_pad = NS * n_pt * CH
    pad = E_pad - E
    if pad:
        src = jnp.concatenate([src, jnp.zeros((pad,), jnp.int32)])
        dst = jnp.concatenate([dst, jnp.zeros((pad,), jnp.int32)])
        w = jnp.concatenate([w, jnp.zeros((pad,), jnp.float32)])
    src = src.reshape(NS, n_pt, CH)
    dst = dst.reshape(NS, n_pt, CH)
    w = w.reshape(NS, n_pt, CH)
    rows_per_tile = ((N + NS * 8 - 1) // (NS * 8)) * 8
    N_pad = rows_per_tile * NS
    zeros_init = jnp.zeros((N_pad, D), jnp.float32)
    nvec = D // 16

    mesh = plsc.VectorSubcoreMesh(core_axis_name="c", subcore_axis_name="s")

    @functools.partial(
        pl.kernel,
        out_type=[
            jax.ShapeDtypeStruct((N_pad, D), jnp.float32),
            jax.ShapeDtypeStruct((N_pad, D), jnp.float32),
        ],
        mesh=mesh,
        scratch_types=[
            pltpu.VMEM_SHARED((N_pad, D), jnp.float32),
            pltpu.VMEM((4, CH), jnp.int32),
            pltpu.VMEM((4, CH), jnp.int32),
            pltpu.VMEM((2, CH), jnp.float32),
            pltpu.VMEM((2, CH, D), jnp.float32),
            pltpu.VMEM((2, CH, D), jnp.float32),
            pltpu.SemaphoreType.DMA,
            pltpu.SemaphoreType.DMA,
            pltpu.SemaphoreType.DMA,
            pltpu.SemaphoreType.DMA,
            pltpu.SemaphoreType.DMA,
            pltpu.SemaphoreType.DMA,
            pltpu.SemaphoreType.DMA,
        ],
    )
    def sc_kernel(y_hbm, src_hbm, dst_hbm, w_hbm, zero_hbm, p0_hbm, p1_hbm,
                  acc, src_r, dst_r, w_r, rows_in, rows_out,
                  gsem0, gsem1, ssem0, ssem1, srcsem, dstsem, wsem):
        c = lax.axis_index("c")
        s = lax.axis_index("s")
        cbase = jnp.where(c == 0, 0, n0)
        n_ch = jnp.where(c == 0, n0, n1)
        tile_rows = pl.ds(s * rows_per_tile, rows_per_tile)
        gsem = (gsem0, gsem1)
        ssem = (ssem0, ssem1)

        def load_src(b, ring_slot):
            pltpu.make_async_copy(
                src_hbm.at[s, cbase + b], src_r.at[ring_slot], srcsem).start()

        def load_dst(b, ring_slot):
            pltpu.make_async_copy(
                dst_hbm.at[s, cbase + b], dst_r.at[ring_slot], dstsem).start()

        def load_w(b, slot):
            pltpu.make_async_copy(
                w_hbm.at[s, cbase + b], w_r.at[slot], wsem).start()

        pltpu.sync_copy(zero_hbm.at[tile_rows], acc.at[tile_rows])

        for b in range(4):
            load_src(b, b)
            load_dst(b, b)
        for slot in range(2):
            load_w(slot, slot)
        plsc.subcore_barrier()

        for slot in range(2):
            pltpu.make_async_copy(
                src_hbm.at[s, cbase + slot], src_r.at[slot], srcsem).wait()
            pltpu.make_async_copy(
                y_hbm.at[src_r.at[slot]], rows_in.at[slot], gsem[slot]).start()

        n2 = n_ch // 2

        def chunk_pair(b2, carry):
            r2 = lax.rem(b2, 2)
            for slot in range(2):
                b = 2 * b2 + slot
                i4 = 2 * r2 + slot
                i4n = 2 * (1 - r2) + slot
                rin = rows_in.at[slot]
                rout = rows_out.at[slot]
                pltpu.make_async_copy(
                    y_hbm.at[src_r.at[i4]], rin, gsem[slot]
                ).wait()

                @pl.when(b2 >= 1)
                def _():
                    pltpu.make_async_copy(
                        rout, acc.at[dst_r.at[i4n]], ssem[slot]
                    ).wait()

                @pl.when(jnp.logical_and(b2 >= 1, b + 2 < n_ch))
                def _():
                    load_dst(b + 2, i4n)

                pltpu.make_async_copy(
                    w_hbm.at[s, cbase + b], w_r.at[slot], wsem
                ).wait()

                for g in range(CH // 16):
                    wv = w_r[slot, pl.ds(g * 16, 16)]
                    for t in range(16):
                        e = g * 16 + t
                        w_e = wv[t]
                        for j in range(nvec):
                            sl = pl.ds(j * 16, 16)
                            rout[e, sl] = rin[e, sl] * w_e

                @pl.when(b + 2 < n_ch)
                def _():
                    load_w(b + 2, slot)

                @pl.when(b + 4 < n_ch)
                def _():
                    load_src(b + 4, i4)

                @pl.when(b + 2 < n_ch)
                def _():
                    pltpu.make_async_copy(
                        src_hbm.at[s, cbase + b + 2], src_r.at[i4n], srcsem).wait()
                    pltpu.make_async_copy(
                        y_hbm.at[src_r.at[i4n]], rin, gsem[slot]).start()

                pltpu.make_async_copy(
                    dst_hbm.at[s, cbase + b], dst_r.at[i4], dstsem).wait()
                pltpu.make_async_copy(
                    rout, acc.at[dst_r.at[i4]], ssem[slot]
                ).start(add=True)
            return carry

        lax.fori_loop(0, n2, chunk_pair, 0)
        for slot in range(2):
            pltpu.make_async_copy(
                rows_out.at[slot], acc.at[dst_r.at[slot]], ssem[slot]
            ).wait()
        plsc.subcore_barrier()

        @pl.when(c == 0)
        def _():
            pltpu.sync_copy(acc.at[tile_rows], p0_hbm.at[tile_rows])

        @pl.when(c == 1)
        def _():
            pltpu.sync_copy(acc.at[tile_rows], p1_hbm.at[tile_rows])

    return sc_kernel(y, src, dst, w, zeros_init)


def _post_body(z_ref, p0_ref, p1_ref, g_ref, b_ref, o_ref):
    h = z_ref[...] + p0_ref[...] + p1_ref[...]
    h = jnp.maximum(h, 0.0)
    mean = jnp.mean(h, axis=1, keepdims=True)
    cen = h - mean
    var = jnp.mean(cen * cen, axis=1, keepdims=True)
    o_ref[...] = cen * lax.rsqrt(var + 1e-5) * g_ref[...] + b_ref[...]


def _post(z, p0, p1, gamma, beta):
    N, D = z.shape
    B = 2000 if N % 2000 == 0 else 8
    grid = (N // B,)
    return pl.pallas_call(
        _post_body,
        grid=grid,
        in_specs=[
            pl.BlockSpec((B, D), lambda i: (i, 0)),
            pl.BlockSpec((B, D), lambda i: (i, 0)),
            pl.BlockSpec((B, D), lambda i: (i, 0)),
            pl.BlockSpec((1, D), lambda i: (0, 0)),
            pl.BlockSpec((1, D), lambda i: (0, 0)),
        ],
        out_specs=pl.BlockSpec((B, D), lambda i: (i, 0)),
        out_shape=jax.ShapeDtypeStruct((N, D), jnp.float32),
    )(z, p0, p1, gamma.reshape(1, D), beta.reshape(1, D))


def kernel(x, edgeIndex, edgeWeight, W_self, b_self, W_neigh, b_neigh, gamma, beta):
    y, z = _prep(x, W_neigh, W_self, b_self, b_neigh)
    p0, p1 = _scatter_sc(y, edgeIndex[0], edgeIndex[1], edgeWeight)
    return _post(z, p0, p1, gamma, beta)


# speedup vs baseline: 9.4362x; 1.0429x over previous
import functools

import jax
import jax.numpy as jnp
from jax import lax
from jax.experimental import pallas as pl
from jax.experimental.pallas import tpu as pltpu
from jax.experimental.pallas import tpu_sc as plsc

NC = 2
NS = 16
NW = NC * NS
CH = 80


FRAC0 = 0.62


def _scatter_sc(y, src, dst, w):
    N, D = y.shape
    E = src.shape[0]
    n_pt = ((E + 2 * NS * CH - 1) // (2 * NS * CH)) * 2
    n0 = max(4, int(round(FRAC0 * n_pt / 2)) * 2)
    n1 = n_pt - n0
    assert n1 >= 4
    E_pad = NS * n_pt * CH
    pad = E_pad - E
    if pad:
        src = jnp.concatenate([src, jnp.zeros((pad,), jnp.int32)])
        dst = jnp.concatenate([dst, jnp.zeros((pad,), jnp.int32)])
        w = jnp.concatenate([w, jnp.zeros((pad,), jnp.float32)])
    src = src.reshape(NS, n_pt, CH)
    dst = dst.reshape(NS, n_pt, CH)
    w = w.reshape(NS, n_pt, CH)
    rows_per_tile = ((N + NS * 8 - 1) // (NS * 8)) * 8
    N_pad = rows_per_tile * NS
    zeros_init = jnp.zeros((N_pad, D), jnp.float32)
    nvec = D // 16

    mesh = plsc.VectorSubcoreMesh(core_axis_name="c", subcore_axis_name="s")

    @functools.partial(
        pl.kernel,
        out_type=[
            jax.ShapeDtypeStruct((N_pad, D), jnp.float32),
            jax.ShapeDtypeStruct((N_pad, D), jnp.float32),
        ],
        mesh=mesh,
        scratch_types=[
            pltpu.VMEM_SHARED((N_pad, D), jnp.float32),
            pltpu.VMEM((4, CH), jnp.int32),
            pltpu.VMEM((4, CH), jnp.int32),
            pltpu.VMEM((2, CH), jnp.float32),
            pltpu.VMEM((2, CH, D), jnp.float32),
            pltpu.VMEM((2, CH, D), jnp.float32),
            pltpu.SemaphoreType.DMA,
            pltpu.SemaphoreType.DMA,
            pltpu.SemaphoreType.DMA,
            pltpu.SemaphoreType.DMA,
            pltpu.SemaphoreType.DMA,
            pltpu.SemaphoreType.DMA,
            pltpu.SemaphoreType.DMA,
        ],
    )
    def sc_kernel(y_hbm, src_hbm, dst_hbm, w_hbm, zero_hbm, p0_hbm, p1_hbm,
                  acc, src_r, dst_r, w_r, rows_in, rows_out,
                  gsem0, gsem1, ssem0, ssem1, srcsem, dstsem, wsem):
        c = lax.axis_index("c")
        s = lax.axis_index("s")
        cbase = jnp.where(c == 0, 0, n0)
        n_ch = jnp.where(c == 0, n0, n1)
        tile_rows = pl.ds(s * rows_per_tile, rows_per_tile)
        gsem = (gsem0, gsem1)
        ssem = (ssem0, ssem1)

        def load_src(b, ring_slot):
            pltpu.make_async_copy(
                src_hbm.at[s, cbase + b], src_r.at[ring_slot], srcsem).start()

        def load_dst(b, ring_slot):
            pltpu.make_async_copy(
                dst_hbm.at[s, cbase + b], dst_r.at[ring_slot], dstsem).start()

        def load_w(b, slot):
            pltpu.make_async_copy(
                w_hbm.at[s, cbase + b], w_r.at[slot], wsem).start()

        pltpu.sync_copy(zero_hbm.at[tile_rows], acc.at[tile_rows])

        for b in range(4):
            load_src(b, b)
            load_dst(b, b)
        for slot in range(2):
            load_w(slot, slot)
        plsc.subcore_barrier()

        for slot in range(2):
            pltpu.make_async_copy(
                src_hbm.at[s, cbase + slot], src_r.at[slot], srcsem).wait()
            pltpu.make_async_copy(
                y_hbm.at[src_r.at[slot]], rows_in.at[slot], gsem[slot]).start()

        n2 = n_ch // 2

        def chunk_pair(b2, carry):
            r2 = lax.rem(b2, 2)
            for slot in range(2):
                b = 2 * b2 + slot
                i4 = 2 * r2 + slot
                i4n = 2 * (1 - r2) + slot
                rin = rows_in.at[slot]
                rout = rows_out.at[slot]
                pltpu.make_async_copy(
                    y_hbm.at[src_r.at[i4]], rin, gsem[slot]
                ).wait()

                @pl.when(b2 >= 1)
                def _():
                    pltpu.make_async_copy(
                        rout, acc.at[dst_r.at[i4n]], ssem[slot]
                    ).wait()

                @pl.when(jnp.logical_and(b2 >= 1, b + 2 < n_ch))
                def _():
                    load_dst(b + 2, i4n)

                pltpu.make_async_copy(
                    w_hbm.at[s, cbase + b], w_r.at[slot], wsem
                ).wait()

                for g in range(CH // 16):
                    wv = w_r[slot, pl.ds(g * 16, 16)]
                    for t in range(16):
                        e = g * 16 + t
                        w_e = wv[t]
                        for j in range(nvec):
                            sl = pl.ds(j * 16, 16)
                            rout[e, sl] = rin[e, sl] * w_e

                @pl.when(b + 2 < n_ch)
                def _():
                    load_w(b + 2, slot)

                @pl.when(b + 4 < n_ch)
                def _():
                    load_src(b + 4, i4)

                @pl.when(b + 2 < n_ch)
                def _():
                    pltpu.make_async_copy(
                        src_hbm.at[s, cbase + b + 2], src_r.at[i4n], srcsem).wait()
                    pltpu.make_async_copy(
                        y_hbm.at[src_r.at[i4n]], rin, gsem[slot]).start()

                pltpu.make_async_copy(
                    dst_hbm.at[s, cbase + b], dst_r.at[i4], dstsem).wait()
                pltpu.make_async_copy(
                    rout, acc.at[dst_r.at[i4]], ssem[slot]
                ).start(add=True)
            return carry

        lax.fori_loop(0, n2, chunk_pair, 0)
        for slot in range(2):
            pltpu.make_async_copy(
                rows_out.at[slot], acc.at[dst_r.at[slot]], ssem[slot]
            ).wait()
        plsc.subcore_barrier()

        @pl.when(c == 0)
        def _():
            pltpu.sync_copy(acc.at[tile_rows], p0_hbm.at[tile_rows])

        @pl.when(c == 1)
        def _():
            pltpu.sync_copy(acc.at[tile_rows], p1_hbm.at[tile_rows])

    return sc_kernel(y, src, dst, w, zeros_init)


def _post_body(x_ref, p0_ref, p1_ref, ws_ref, wn_ref, bias_ref, g_ref, b_ref,
               o_ref):
    dn = (((1,), (1,)), ((), ()))
    h = lax.dot_general(x_ref[...], ws_ref[...], dn,
                        preferred_element_type=jnp.float32)
    h = h + lax.dot_general(p0_ref[...] + p1_ref[...], wn_ref[...], dn,
                            preferred_element_type=jnp.float32)
    h = h + bias_ref[...]
    h = jnp.maximum(h, 0.0)
    mean = jnp.mean(h, axis=1, keepdims=True)
    cen = h - mean
    var = jnp.mean(cen * cen, axis=1, keepdims=True)
    o_ref[...] = cen * lax.rsqrt(var + 1e-5) * g_ref[...] + b_ref[...]


def _post(x, p0, p1, W_self, W_neigh, b_self, b_neigh, gamma, beta):
    N, D = x.shape
    B = 2000 if N % 2000 == 0 else 8
    grid = (N // B,)
    bias = (b_self + b_neigh).reshape(1, D)
    return pl.pallas_call(
        _post_body,
        grid=grid,
        in_specs=[
            pl.BlockSpec((B, D), lambda i: (i, 0)),
            pl.BlockSpec((B, D), lambda i: (i, 0)),
            pl.BlockSpec((B, D), lambda i: (i, 0)),
            pl.BlockSpec((D, D), lambda i: (0, 0)),
            pl.BlockSpec((D, D), lambda i: (0, 0)),
            pl.BlockSpec((1, D), lambda i: (0, 0)),
            pl.BlockSpec((1, D), lambda i: (0, 0)),
            pl.BlockSpec((1, D), lambda i: (0, 0)),
        ],
        out_specs=pl.BlockSpec((B, D), lambda i: (i, 0)),
        out_shape=jax.ShapeDtypeStruct((N, D), jnp.float32),
    )(x, p0, p1, W_self, W_neigh, bias, gamma.reshape(1, D),
      beta.reshape(1, D))


def kernel(x, edgeIndex, edgeWeight, W_self, b_self, W_neigh, b_neigh, gamma, beta):
    p0, p1 = _scatter_sc(x, edgeIndex[0], edgeIndex[1], edgeWeight)
    return _post(x, p0, p1, W_self, W_neigh, b_self, b_neigh, gamma, beta)


# speedup vs baseline: 10.1241x; 1.0729x over previous
import functools

import jax
import jax.numpy as jnp
from jax import lax
from jax.experimental import pallas as pl
from jax.experimental.pallas import tpu as pltpu
from jax.experimental.pallas import tpu_sc as plsc

NC = 2
NS = 16
NW = NC * NS
CH = 80


FRAC0 = 0.56


def _scatter_sc(y, src, dst, w):
    N, D = y.shape
    E = src.shape[0]
    n_pt = ((E + 2 * NS * CH - 1) // (2 * NS * CH)) * 2
    n0 = max(4, int(round(FRAC0 * n_pt / 2)) * 2)
    n1 = n_pt - n0
    assert n1 >= 4
    E_pad = NS * n_pt * CH
    pad = E_pad - E
    if pad:
        src = jnp.concatenate([src, jnp.zeros((pad,), jnp.int32)])
        dst = jnp.concatenate([dst, jnp.zeros((pad,), jnp.int32)])
        w = jnp.concatenate([w, jnp.zeros((pad,), jnp.float32)])
    src = src.reshape(NS, n_pt, CH)
    dst = dst.reshape(NS, n_pt, CH)
    w = w.reshape(NS, n_pt, CH)
    rows_per_tile = ((N + NS * 8 - 1) // (NS * 8)) * 8
    N_pad = rows_per_tile * NS
    zeros_init = jnp.zeros((N_pad, D), jnp.float32)
    nvec = D // 16

    mesh = plsc.VectorSubcoreMesh(core_axis_name="c", subcore_axis_name="s")

    @functools.partial(
        pl.kernel,
        out_type=[
            jax.ShapeDtypeStruct((N_pad, D), jnp.float32),
            jax.ShapeDtypeStruct((N_pad, D), jnp.float32),
        ],
        mesh=mesh,
        scratch_types=[
            pltpu.VMEM_SHARED((N_pad, D), jnp.float32),
            pltpu.VMEM((4, CH), jnp.int32),
            pltpu.VMEM((4, CH), jnp.int32),
            pltpu.VMEM((2, CH), jnp.float32),
            pltpu.VMEM((2, CH, D), jnp.float32),
            pltpu.VMEM((2, CH, D), jnp.float32),
            pltpu.SemaphoreType.DMA,
            pltpu.SemaphoreType.DMA,
            pltpu.SemaphoreType.DMA,
            pltpu.SemaphoreType.DMA,
            pltpu.SemaphoreType.DMA,
            pltpu.SemaphoreType.DMA,
            pltpu.SemaphoreType.DMA,
        ],
    )
    def sc_kernel(y_hbm, src_hbm, dst_hbm, w_hbm, zero_hbm, p0_hbm, p1_hbm,
                  acc, src_r, dst_r, w_r, rows_in, rows_out,
                  gsem0, gsem1, ssem0, ssem1, srcsem, dstsem, wsem):
        c = lax.axis_index("c")
        s = lax.axis_index("s")
        cbase = jnp.where(c == 0, 0, n0)
        n_ch = jnp.where(c == 0, n0, n1)
        tile_rows = pl.ds(s * rows_per_tile, rows_per_tile)
        gsem = (gsem0, gsem1)
        ssem = (ssem0, ssem1)

        def load_src(b, ring_slot):
            pltpu.make_async_copy(
                src_hbm.at[s, cbase + b], src_r.at[ring_slot], srcsem).start()

        def load_dst(b, ring_slot):
            pltpu.make_async_copy(
                dst_hbm.at[s, cbase + b], dst_r.at[ring_slot], dstsem).start()

        def load_w(b, slot):
            pltpu.make_async_copy(
                w_hbm.at[s, cbase + b], w_r.at[slot], wsem).start()

        pltpu.sync_copy(zero_hbm.at[tile_rows], acc.at[tile_rows])

        for b in range(4):
            load_src(b, b)
            load_dst(b, b)
        for slot in range(2):
            load_w(slot, slot)
        plsc.subcore_barrier()

        for slot in range(2):
            pltpu.make_async_copy(
                src_hbm.at[s, cbase + slot], src_r.at[slot], srcsem).wait()
            pltpu.make_async_copy(
                y_hbm.at[src_r.at[slot]], rows_in.at[slot], gsem[slot]).start()

        n2 = n_ch // 2

        def chunk_pair(b2, carry):
            r2 = lax.rem(b2, 2)
            for slot in range(2):
                b = 2 * b2 + slot
                i4 = 2 * r2 + slot
                i4n = 2 * (1 - r2) + slot
                rin = rows_in.at[slot]
                rout = rows_out.at[slot]
                pltpu.make_async_copy(
                    y_hbm.at[src_r.at[i4]], rin, gsem[slot]
                ).wait()

                @pl.when(b2 >= 1)
                def _():
                    pltpu.make_async_copy(
                        rout, acc.at[dst_r.at[i4n]], ssem[slot]
                    ).wait()

                @pl.when(jnp.logical_and(b2 >= 1, b + 2 < n_ch))
                def _():
                    load_dst(b + 2, i4n)

                pltpu.make_async_copy(
                    w_hbm.at[s, cbase + b], w_r.at[slot], wsem
                ).wait()

                for g in range(CH // 16):
                    wv = w_r[slot, pl.ds(g * 16, 16)]
                    for t in range(16):
                        e = g * 16 + t
                        w_e = wv[t]
                        for j in range(nvec):
                            sl = pl.ds(j * 16, 16)
                            rout[e, sl] = rin[e, sl] * w_e

                @pl.when(b + 2 < n_ch)
                def _():
                    load_w(b + 2, slot)

                @pl.when(b + 4 < n_ch)
                def _():
                    load_src(b + 4, i4)

                @pl.when(b + 2 < n_ch)
                def _():
                    pltpu.make_async_copy(
                        src_hbm.at[s, cbase + b + 2], src_r.at[i4n], srcsem).wait()
                    pltpu.make_async_copy(
                        y_hbm.at[src_r.at[i4n]], rin, gsem[slot]).start()

                pltpu.make_async_copy(
                    dst_hbm.at[s, cbase + b], dst_r.at[i4], dstsem).wait()
                pltpu.make_async_copy(
                    rout, acc.at[dst_r.at[i4]], ssem[slot]
                ).start(add=True)
            return carry

        lax.fori_loop(0, n2, chunk_pair, 0)
        for slot in range(2):
            pltpu.make_async_copy(
                rows_out.at[slot], acc.at[dst_r.at[slot]], ssem[slot]
            ).wait()
        plsc.subcore_barrier()

        @pl.when(c == 0)
        def _():
            pltpu.sync_copy(acc.at[tile_rows], p0_hbm.at[tile_rows])

        @pl.when(c == 1)
        def _():
            pltpu.sync_copy(acc.at[tile_rows], p1_hbm.at[tile_rows])

    return sc_kernel(y, src, dst, w, zeros_init)


def _post_body(x_ref, p0_ref, p1_ref, ws_ref, wn_ref, bias_ref, g_ref, b_ref,
               o_ref):
    dn = (((1,), (1,)), ((), ()))
    h = lax.dot_general(x_ref[...], ws_ref[...], dn,
                        preferred_element_type=jnp.float32)
    h = h + lax.dot_general(p0_ref[...] + p1_ref[...], wn_ref[...], dn,
                            preferred_element_type=jnp.float32)
    h = h + bias_ref[...]
    h = jnp.maximum(h, 0.0)
    mean = jnp.mean(h, axis=1, keepdims=True)
    cen = h - mean
    var = jnp.mean(cen * cen, axis=1, keepdims=True)
    o_ref[...] = cen * lax.rsqrt(var + 1e-5) * g_ref[...] + b_ref[...]


def _post(x, p0, p1, W_self, W_neigh, b_self, b_neigh, gamma, beta):
    N, D = x.shape
    B = 2000 if N % 2000 == 0 else 8
    grid = (N // B,)
    bias = (b_self + b_neigh).reshape(1, D)
    return pl.pallas_call(
        _post_body,
        grid=grid,
        in_specs=[
            pl.BlockSpec((B, D), lambda i: (i, 0)),
            pl.BlockSpec((B, D), lambda i: (i, 0)),
            pl.BlockSpec((B, D), lambda i: (i, 0)),
            pl.BlockSpec((D, D), lambda i: (0, 0)),
            pl.BlockSpec((D, D), lambda i: (0, 0)),
            pl.BlockSpec((1, D), lambda i: (0, 0)),
            pl.BlockSpec((1, D), lambda i: (0, 0)),
            pl.BlockSpec((1, D), lambda i: (0, 0)),
        ],
        out_specs=pl.BlockSpec((B, D), lambda i: (i, 0)),
        out_shape=jax.ShapeDtypeStruct((N, D), jnp.float32),
    )(x, p0, p1, W_self, W_neigh, bias, gamma.reshape(1, D),
      beta.reshape(1, D))


def kernel(x, edgeIndex, edgeWeight, W_self, b_self, W_neigh, b_neigh, gamma, beta):
    p0, p1 = _scatter_sc(x, edgeIndex[0], edgeIndex[1], edgeWeight)
    return _post(x, p0, p1, W_self, W_neigh, b_self, b_neigh, gamma, beta)


# speedup vs baseline: 10.5942x; 1.0464x over previous
import functools

import jax
import jax.numpy as jnp
from jax import lax
from jax.experimental import pallas as pl
from jax.experimental.pallas import tpu as pltpu
from jax.experimental.pallas import tpu_sc as plsc

NC = 2
NS = 16
NW = NC * NS
CH = 80


FRAC0 = 0.52


def _scatter_sc(y, src, dst, w):
    N, D = y.shape
    E = src.shape[0]
    n_pt = ((E + 2 * NS * CH - 1) // (2 * NS * CH)) * 2
    n0 = max(4, int(round(FRAC0 * n_pt / 2)) * 2)
    n1 = n_pt - n0
    assert n1 >= 4
    E_pad = NS * n_pt * CH
    pad = E_pad - E
    if pad:
        src = jnp.concatenate([src, jnp.zeros((pad,), jnp.int32)])
        dst = jnp.concatenate([dst, jnp.zeros((pad,), jnp.int32)])
        w = jnp.concatenate([w, jnp.zeros((pad,), jnp.float32)])
    src = src.reshape(NS, n_pt, CH)
    dst = dst.reshape(NS, n_pt, CH)
    w = w.reshape(NS, n_pt, CH)
    rows_per_tile = ((N + NS * 8 - 1) // (NS * 8)) * 8
    N_pad = rows_per_tile * NS
    zeros_init = jnp.zeros((N_pad, D), jnp.float32)
    nvec = D // 16

    mesh = plsc.VectorSubcoreMesh(core_axis_name="c", subcore_axis_name="s")

    @functools.partial(
        pl.kernel,
        out_type=[
            jax.ShapeDtypeStruct((N_pad, D), jnp.float32),
            jax.ShapeDtypeStruct((N_pad, D), jnp.float32),
        ],
        mesh=mesh,
        scratch_types=[
            pltpu.VMEM_SHARED((N_pad, D), jnp.float32),
            pltpu.VMEM((4, CH), jnp.int32),
            pltpu.VMEM((4, CH), jnp.int32),
            pltpu.VMEM((2, CH), jnp.float32),
            pltpu.VMEM((2, CH, D), jnp.float32),
            pltpu.VMEM((2, CH, D), jnp.float32),
            pltpu.SemaphoreType.DMA,
            pltpu.SemaphoreType.DMA,
            pltpu.SemaphoreType.DMA,
            pltpu.SemaphoreType.DMA,
            pltpu.SemaphoreType.DMA,
            pltpu.SemaphoreType.DMA,
            pltpu.SemaphoreType.DMA,
        ],
    )
    def sc_kernel(y_hbm, src_hbm, dst_hbm, w_hbm, zero_hbm, p0_hbm, p1_hbm,
                  acc, src_r, dst_r, w_r, rows_in, rows_out,
                  gsem0, gsem1, ssem0, ssem1, srcsem, dstsem, wsem):
        c = lax.axis_index("c")
        s = lax.axis_index("s")
        cbase = jnp.where(c == 0, 0, n0)
        n_ch = jnp.where(c == 0, n0, n1)
        tile_rows = pl.ds(s * rows_per_tile, rows_per_tile)
        gsem = (gsem0, gsem1)
        ssem = (ssem0, ssem1)

        def load_src(b, ring_slot):
            pltpu.make_async_copy(
                src_hbm.at[s, cbase + b], src_r.at[ring_slot], srcsem).start()

        def load_dst(b, ring_slot):
            pltpu.make_async_copy(
                dst_hbm.at[s, cbase + b], dst_r.at[ring_slot], dstsem).start()

        def load_w(b, slot):
            pltpu.make_async_copy(
                w_hbm.at[s, cbase + b], w_r.at[slot], wsem).start()

        pltpu.sync_copy(zero_hbm.at[tile_rows], acc.at[tile_rows])

        for b in range(4):
            load_src(b, b)
            load_dst(b, b)
        for slot in range(2):
            load_w(slot, slot)
        plsc.subcore_barrier()

        for slot in range(2):
            pltpu.make_async_copy(
                src_hbm.at[s, cbase + slot], src_r.at[slot], srcsem).wait()
            pltpu.make_async_copy(
                y_hbm.at[src_r.at[slot]], rows_in.at[slot], gsem[slot]).start()

        n2 = n_ch // 2

        def chunk_pair(b2, carry):
            r2 = lax.rem(b2, 2)
            for slot in range(2):
                b = 2 * b2 + slot
                i4 = 2 * r2 + slot
                i4n = 2 * (1 - r2) + slot
                rin = rows_in.at[slot]
                rout = rows_out.at[slot]
                pltpu.make_async_copy(
                    y_hbm.at[src_r.at[i4]], rin, gsem[slot]
                ).wait()

                @pl.when(b2 >= 1)
                def _():
                    pltpu.make_async_copy(
                        rout, acc.at[dst_r.at[i4n]], ssem[slot]
                    ).wait()

                @pl.when(jnp.logical_and(b2 >= 1, b + 2 < n_ch))
                def _():
                    load_dst(b + 2, i4n)

                pltpu.make_async_copy(
                    w_hbm.at[s, cbase + b], w_r.at[slot], wsem
                ).wait()

                for g in range(CH // 16):
                    wv = w_r[slot, pl.ds(g * 16, 16)]
                    for t in range(16):
                        e = g * 16 + t
                        w_e = wv[t]
                        for j in range(nvec):
                            sl = pl.ds(j * 16, 16)
                            rout[e, sl] = rin[e, sl] * w_e

                @pl.when(b + 2 < n_ch)
                def _():
                    load_w(b + 2, slot)

                @pl.when(b + 4 < n_ch)
                def _():
                    load_src(b + 4, i4)

                @pl.when(b + 2 < n_ch)
                def _():
                    pltpu.make_async_copy(
                        src_hbm.at[s, cbase + b + 2], src_r.at[i4n], srcsem).wait()
                    pltpu.make_async_copy(
                        y_hbm.at[src_r.at[i4n]], rin, gsem[slot]).start()

                pltpu.make_async_copy(
                    dst_hbm.at[s, cbase + b], dst_r.at[i4], dstsem).wait()
                pltpu.make_async_copy(
                    rout, acc.at[dst_r.at[i4]], ssem[slot]
                ).start(add=True)
            return carry

        lax.fori_loop(0, n2, chunk_pair, 0)
        for slot in range(2):
            pltpu.make_async_copy(
                rows_out.at[slot], acc.at[dst_r.at[slot]], ssem[slot]
            ).wait()
        plsc.subcore_barrier()

        @pl.when(c == 0)
        def _():
            pltpu.sync_copy(acc.at[tile_rows], p0_hbm.at[tile_rows])

        @pl.when(c == 1)
        def _():
            pltpu.sync_copy(acc.at[tile_rows], p1_hbm.at[tile_rows])

    return sc_kernel(y, src, dst, w, zeros_init)


def _post_body(x_ref, p0_ref, p1_ref, ws_ref, wn_ref, bias_ref, g_ref, b_ref,
               o_ref):
    dn = (((1,), (1,)), ((), ()))
    h = lax.dot_general(x_ref[...], ws_ref[...], dn,
                        preferred_element_type=jnp.float32)
    h = h + lax.dot_general(p0_ref[...] + p1_ref[...], wn_ref[...], dn,
                            preferred_element_type=jnp.float32)
    h = h + bias_ref[...]
    h = jnp.maximum(h, 0.0)
    mean = jnp.mean(h, axis=1, keepdims=True)
    cen = h - mean
    var = jnp.mean(cen * cen, axis=1, keepdims=True)
    o_ref[...] = cen * lax.rsqrt(var + 1e-5) * g_ref[...] + b_ref[...]


def _post(x, p0, p1, W_self, W_neigh, b_self, b_neigh, gamma, beta):
    N, D = x.shape
    B = 2000 if N % 2000 == 0 else 8
    grid = (N // B,)
    bias = (b_self + b_neigh).reshape(1, D)
    return pl.pallas_call(
        _post_body,
        grid=grid,
        in_specs=[
            pl.BlockSpec((B, D), lambda i: (i, 0)),
            pl.BlockSpec((B, D), lambda i: (i, 0)),
            pl.BlockSpec((B, D), lambda i: (i, 0)),
            pl.BlockSpec((D, D), lambda i: (0, 0)),
            pl.BlockSpec((D, D), lambda i: (0, 0)),
            pl.BlockSpec((1, D), lambda i: (0, 0)),
            pl.BlockSpec((1, D), lambda i: (0, 0)),
            pl.BlockSpec((1, D), lambda i: (0, 0)),
        ],
        out_specs=pl.BlockSpec((B, D), lambda i: (i, 0)),
        out_shape=jax.ShapeDtypeStruct((N, D), jnp.float32),
    )(x, p0, p1, W_self, W_neigh, bias, gamma.reshape(1, D),
      beta.reshape(1, D))


def kernel(x, edgeIndex, edgeWeight, W_self, b_self, W_neigh, b_neigh, gamma, beta):
    p0, p1 = _scatter_sc(x, edgeIndex[0], edgeIndex[1], edgeWeight)
    return _post(x, p0, p1, W_self, W_neigh, b_self, b_neigh, gamma, beta)


# speedup vs baseline: 10.7883x; 1.0183x over previous
import functools

import jax
import jax.numpy as jnp
from jax import lax
from jax.experimental import pallas as pl
from jax.experimental.pallas import tpu as pltpu
from jax.experimental.pallas import tpu_sc as plsc

NC = 2
NS = 16
NW = NC * NS
CH = 80


FRAC0 = 0.50


def _scatter_sc(y, src, dst, w):
    N, D = y.shape
    E = src.shape[0]
    n_pt = ((E + 2 * NS * CH - 1) // (2 * NS * CH)) * 2
    n0 = max(4, int(round(FRAC0 * n_pt / 2)) * 2)
    n1 = n_pt - n0
    assert n1 >= 4
    E_pad = NS * n_pt * CH
    pad = E_pad - E
    if pad:
        src = jnp.concatenate([src, jnp.zeros((pad,), jnp.int32)])
        dst = jnp.concatenate([dst, jnp.zeros((pad,), jnp.int32)])
        w = jnp.concatenate([w, jnp.zeros((pad,), jnp.float32)])
    src = src.reshape(NS, n_pt, CH)
    dst = dst.reshape(NS, n_pt, CH)
    w = w.reshape(NS, n_pt, CH)
    rows_per_tile = ((N + NS * 8 - 1) // (NS * 8)) * 8
    N_pad = rows_per_tile * NS
    zeros_init = jnp.zeros((N_pad, D), jnp.float32)
    nvec = D // 16

    mesh = plsc.VectorSubcoreMesh(core_axis_name="c", subcore_axis_name="s")

    @functools.partial(
        pl.kernel,
        out_type=[
            jax.ShapeDtypeStruct((N_pad, D), jnp.float32),
            jax.ShapeDtypeStruct((N_pad, D), jnp.float32),
        ],
        mesh=mesh,
        scratch_types=[
            pltpu.VMEM_SHARED((N_pad, D), jnp.float32),
            pltpu.VMEM((4, CH), jnp.int32),
            pltpu.VMEM((4, CH), jnp.int32),
            pltpu.VMEM((2, CH), jnp.float32),
            pltpu.VMEM((2, CH, D), jnp.float32),
            pltpu.VMEM((2, CH, D), jnp.float32),
            pltpu.SemaphoreType.DMA,
            pltpu.SemaphoreType.DMA,
            pltpu.SemaphoreType.DMA,
            pltpu.SemaphoreType.DMA,
            pltpu.SemaphoreType.DMA,
            pltpu.SemaphoreType.DMA,
            pltpu.SemaphoreType.DMA,
        ],
    )
    def sc_kernel(y_hbm, src_hbm, dst_hbm, w_hbm, zero_hbm, p0_hbm, p1_hbm,
                  acc, src_r, dst_r, w_r, rows_in, rows_out,
                  gsem0, gsem1, ssem0, ssem1, srcsem, dstsem, wsem):
        c = lax.axis_index("c")
        s = lax.axis_index("s")
        cbase = jnp.where(c == 0, 0, n0)
        n_ch = jnp.where(c == 0, n0, n1)
        tile_rows = pl.ds(s * rows_per_tile, rows_per_tile)
        gsem = (gsem0, gsem1)
        ssem = (ssem0, ssem1)

        def load_src(b, ring_slot):
            pltpu.make_async_copy(
                src_hbm.at[s, cbase + b], src_r.at[ring_slot], srcsem).start()

        def load_dst(b, ring_slot):
            pltpu.make_async_copy(
                dst_hbm.at[s, cbase + b], dst_r.at[ring_slot], dstsem).start()

        def load_w(b, slot):
            pltpu.make_async_copy(
                w_hbm.at[s, cbase + b], w_r.at[slot], wsem).start()

        pltpu.sync_copy(zero_hbm.at[tile_rows], acc.at[tile_rows])

        for b in range(4):
            load_src(b, b)
            load_dst(b, b)
        for slot in range(2):
            load_w(slot, slot)
        plsc.subcore_barrier()

        for slot in range(2):
            pltpu.make_async_copy(
                src_hbm.at[s, cbase + slot], src_r.at[slot], srcsem).wait()
            pltpu.make_async_copy(
                y_hbm.at[src_r.at[slot]], rows_in.at[slot], gsem[slot]).start()

        n2 = n_ch // 2

        def chunk_pair(b2, carry):
            r2 = lax.rem(b2, 2)
            for slot in range(2):
                b = 2 * b2 + slot
                i4 = 2 * r2 + slot
                i4n = 2 * (1 - r2) + slot
                rin = rows_in.at[slot]
                rout = rows_out.at[slot]
                pltpu.make_async_copy(
                    y_hbm.at[src_r.at[i4]], rin, gsem[slot]
                ).wait()

                @pl.when(b2 >= 1)
                def _():
                    pltpu.make_async_copy(
                        rout, acc.at[dst_r.at[i4n]], ssem[slot]
                    ).wait()

                @pl.when(jnp.logical_and(b2 >= 1, b + 2 < n_ch))
                def _():
                    load_dst(b + 2, i4n)

                pltpu.make_async_copy(
                    w_hbm.at[s, cbase + b], w_r.at[slot], wsem
                ).wait()

                for g in range(CH // 16):
                    wv = w_r[slot, pl.ds(g * 16, 16)]
                    for t in range(16):
                        e = g * 16 + t
                        w_e = wv[t]
                        for j in range(nvec):
                            sl = pl.ds(j * 16, 16)
                            rout[e, sl] = rin[e, sl] * w_e

                @pl.when(b + 2 < n_ch)
                def _():
                    load_w(b + 2, slot)

                @pl.when(b + 4 < n_ch)
                def _():
                    load_src(b + 4, i4)

                @pl.when(b + 2 < n_ch)
                def _():
                    pltpu.make_async_copy(
                        src_hbm.at[s, cbase + b + 2], src_r.at[i4n], srcsem).wait()
                    pltpu.make_async_copy(
                        y_hbm.at[src_r.at[i4n]], rin, gsem[slot]).start()

                pltpu.make_async_copy(
                    dst_hbm.at[s, cbase + b], dst_r.at[i4], dstsem).wait()
                pltpu.make_async_copy(
                    rout, acc.at[dst_r.at[i4]], ssem[slot]
                ).start(add=True)
            return carry

        lax.fori_loop(0, n2, chunk_pair, 0)
        for slot in range(2):
            pltpu.make_async_copy(
                rows_out.at[slot], acc.at[dst_r.at[slot]], ssem[slot]
            ).wait()
        plsc.subcore_barrier()

        @pl.when(c == 0)
        def _():
            pltpu.sync_copy(acc.at[tile_rows], p0_hbm.at[tile_rows])

        @pl.when(c == 1)
        def _():
            pltpu.sync_copy(acc.at[tile_rows], p1_hbm.at[tile_rows])

    return sc_kernel(y, src, dst, w, zeros_init)


def _post_body(x_ref, p0_ref, p1_ref, ws_ref, wn_ref, bias_ref, g_ref, b_ref,
               o_ref):
    dn = (((1,), (1,)), ((), ()))
    h = lax.dot_general(x_ref[...], ws_ref[...], dn,
                        preferred_element_type=jnp.float32)
    h = h + lax.dot_general(p0_ref[...] + p1_ref[...], wn_ref[...], dn,
                            preferred_element_type=jnp.float32)
    h = h + bias_ref[...]
    h = jnp.maximum(h, 0.0)
    mean = jnp.mean(h, axis=1, keepdims=True)
    cen = h - mean
    var = jnp.mean(cen * cen, axis=1, keepdims=True)
    o_ref[...] = cen * lax.rsqrt(var + 1e-5) * g_ref[...] + b_ref[...]


def _post(x, p0, p1, W_self, W_neigh, b_self, b_neigh, gamma, beta):
    N, D = x.shape
    B = 2000 if N % 2000 == 0 else 8
    grid = (N // B,)
    bias = (b_self + b_neigh).reshape(1, D)
    return pl.pallas_call(
        _post_body,
        grid=grid,
        in_specs=[
            pl.BlockSpec((B, D), lambda i: (i, 0)),
            pl.BlockSpec((B, D), lambda i: (i, 0)),
            pl.BlockSpec((B, D), lambda i: (i, 0)),
            pl.BlockSpec((D, D), lambda i: (0, 0)),
            pl.BlockSpec((D, D), lambda i: (0, 0)),
            pl.BlockSpec((1, D), lambda i: (0, 0)),
            pl.BlockSpec((1, D), lambda i: (0, 0)),
            pl.BlockSpec((1, D), lambda i: (0, 0)),
        ],
        out_specs=pl.BlockSpec((B, D), lambda i: (i, 0)),
        out_shape=jax.ShapeDtypeStruct((N, D), jnp.float32),
    )(x, p0, p1, W_self, W_neigh, bias, gamma.reshape(1, D),
      beta.reshape(1, D))


def kernel(x, edgeIndex, edgeWeight, W_self, b_self, W_neigh, b_neigh, gamma, beta):
    p0, p1 = _scatter_sc(x, edgeIndex[0], edgeIndex[1], edgeWeight)
    return _post(x, p0, p1, W_self, W_neigh, b_self, b_neigh, gamma, beta)


# speedup vs baseline: 11.3718x; 1.0541x over previous
import functools

import jax
import jax.numpy as jnp
from jax import lax
from jax.experimental import pallas as pl
from jax.experimental.pallas import tpu as pltpu
from jax.experimental.pallas import tpu_sc as plsc

NC = 2
NS = 16
NW = NC * NS
CH = 80


FRAC0 = 0.50


def _scatter_sc(y, src, dst, w):
    N, D = y.shape
    E = src.shape[0]
    n_pt = ((E + 2 * NS * CH - 1) // (2 * NS * CH)) * 2
    n0 = max(4, int(round(FRAC0 * n_pt / 2)) * 2)
    n1 = n_pt - n0
    assert n1 >= 4
    E_pad = NS * n_pt * CH
    pad = E_pad - E
    if pad:
        src = jnp.concatenate([src, jnp.zeros((pad,), jnp.int32)])
        dst = jnp.concatenate([dst, jnp.zeros((pad,), jnp.int32)])
        w = jnp.concatenate([w, jnp.zeros((pad,), jnp.float32)])
    src = src.reshape(NS, n_pt, CH)
    dst = dst.reshape(NS, n_pt, CH)
    w = w.reshape(NS, n_pt, CH)
    rows_per_tile = ((N + NS * 8 - 1) // (NS * 8)) * 8
    N_pad = rows_per_tile * NS
    zeros_init = jnp.zeros((N_pad, D), jnp.float32)
    nvec = D // 16

    mesh = plsc.VectorSubcoreMesh(core_axis_name="c", subcore_axis_name="s")

    @functools.partial(
        pl.kernel,
        out_type=[
            jax.ShapeDtypeStruct((N_pad, D), jnp.float32),
            jax.ShapeDtypeStruct((N_pad, D), jnp.float32),
        ],
        mesh=mesh,
        scratch_types=[
            pltpu.VMEM_SHARED((N_pad, D), jnp.float32),
            pltpu.VMEM((8, CH), jnp.int32),
            pltpu.VMEM((4, CH), jnp.int32),
            pltpu.VMEM((2, CH), jnp.float32),
            pltpu.VMEM((4, CH, D), jnp.float32),
            pltpu.SemaphoreType.DMA,
            pltpu.SemaphoreType.DMA,
            pltpu.SemaphoreType.DMA,
            pltpu.SemaphoreType.DMA,
            pltpu.SemaphoreType.DMA,
        ],
    )
    def sc_kernel(y_hbm, src_hbm, dst_hbm, w_hbm, zero_hbm, p0_hbm, p1_hbm,
                  acc, src_r, dst_r, w_r, rows,
                  gsem, ssem, srcsem, dstsem, wsem):
        c = lax.axis_index("c")
        s = lax.axis_index("s")
        cbase = jnp.where(c == 0, 0, n0)
        n_ch = jnp.where(c == 0, n0, n1)
        tile_rows = pl.ds(s * rows_per_tile, rows_per_tile)

        def load_src(b):
            pltpu.make_async_copy(
                src_hbm.at[s, cbase + b], src_r.at[lax.rem(b, 8)],
                srcsem).start()

        def load_dst(b):
            pltpu.make_async_copy(
                dst_hbm.at[s, cbase + b], dst_r.at[lax.rem(b, 4)],
                dstsem).start()

        def load_w(b, slot):
            pltpu.make_async_copy(
                w_hbm.at[s, cbase + b], w_r.at[slot], wsem).start()

        pltpu.sync_copy(zero_hbm.at[tile_rows], acc.at[tile_rows])

        for b in range(6):
            load_src(b)
        for b in range(4):
            load_dst(b)
        for slot in range(2):
            load_w(slot, slot)
        plsc.subcore_barrier()

        for b in range(3):
            pltpu.make_async_copy(
                src_hbm.at[s, cbase + b], src_r.at[b], srcsem).wait()
            pltpu.make_async_copy(
                y_hbm.at[src_r.at[b]], rows.at[b], gsem).start()

        n2 = n_ch // 2

        def chunk_pair(b2, carry):
            r2 = lax.rem(b2, 2)
            r4 = lax.rem(b2, 4)
            for slot in range(2):
                b = 2 * b2 + slot
                i4 = 2 * r2 + slot
                i8 = 2 * r4 + slot
                i4p3 = lax.rem(i4 + 3, 4)
                i8p3 = lax.rem(i8 + 3, 8)
                pltpu.make_async_copy(
                    y_hbm.at[src_r.at[i8]], rows.at[i4], gsem).wait()
                pltpu.make_async_copy(
                    w_hbm.at[s, cbase + b], w_r.at[slot], wsem).wait()

                for g in range(CH // 16):
                    wv = w_r[slot, pl.ds(g * 16, 16)]
                    for t in range(16):
                        e = g * 16 + t
                        w_e = wv[t]
                        for j in range(nvec):
                            sl = pl.ds(j * 16, 16)
                            rows[i4, e, sl] = rows[i4, e, sl] * w_e

                @pl.when(b + 2 < n_ch)
                def _():
                    load_w(b + 2, slot)

                @pl.when(b >= 1)
                def _():
                    pltpu.make_async_copy(
                        rows.at[i4p3], acc.at[dst_r.at[i4p3]], ssem).wait()

                @pl.when(jnp.logical_and(b >= 1, b + 3 < n_ch))
                def _():
                    load_dst(b + 3)

                @pl.when(b + 6 < n_ch)
                def _():
                    load_src(b + 6)

                @pl.when(b + 3 < n_ch)
                def _():
                    pltpu.make_async_copy(
                        src_hbm.at[s, cbase + b + 3], src_r.at[i8p3],
                        srcsem).wait()
                    pltpu.make_async_copy(
                        y_hbm.at[src_r.at[i8p3]], rows.at[i4p3], gsem).start()

                pltpu.make_async_copy(
                    dst_hbm.at[s, cbase + b], dst_r.at[i4], dstsem).wait()
                pltpu.make_async_copy(
                    rows.at[i4], acc.at[dst_r.at[i4]], ssem
                ).start(add=True)
            return carry

        lax.fori_loop(0, n2, chunk_pair, 0)
        pltpu.make_async_copy(
            rows.at[0], acc.at[dst_r.at[0]], ssem).wait()
        plsc.subcore_barrier()

        @pl.when(c == 0)
        def _():
            pltpu.sync_copy(acc.at[tile_rows], p0_hbm.at[tile_rows])

        @pl.when(c == 1)
        def _():
            pltpu.sync_copy(acc.at[tile_rows], p1_hbm.at[tile_rows])

    return sc_kernel(y, src, dst, w, zeros_init)


def _post_body(x_ref, p0_ref, p1_ref, ws_ref, wn_ref, bias_ref, g_ref, b_ref,
               o_ref):
    dn = (((1,), (1,)), ((), ()))
    h = lax.dot_general(x_ref[...], ws_ref[...], dn,
                        preferred_element_type=jnp.float32)
    h = h + lax.dot_general(p0_ref[...] + p1_ref[...], wn_ref[...], dn,
                            preferred_element_type=jnp.float32)
    h = h + bias_ref[...]
    h = jnp.maximum(h, 0.0)
    mean = jnp.mean(h, axis=1, keepdims=True)
    cen = h - mean
    var = jnp.mean(cen * cen, axis=1, keepdims=True)
    o_ref[...] = cen * lax.rsqrt(var + 1e-5) * g_ref[...] + b_ref[...]


def _post(x, p0, p1, W_self, W_neigh, b_self, b_neigh, gamma, beta):
    N, D = x.shape
    B = 2000 if N % 2000 == 0 else 8
    grid = (N // B,)
    bias = (b_self + b_neigh).reshape(1, D)
    return pl.pallas_call(
        _post_body,
        grid=grid,
        in_specs=[
            pl.BlockSpec((B, D), lambda i: (i, 0)),
            pl.BlockSpec((B, D), lambda i: (i, 0)),
            pl.BlockSpec((B, D), lambda i: (i, 0)),
            pl.BlockSpec((D, D), lambda i: (0, 0)),
            pl.BlockSpec((D, D), lambda i: (0, 0)),
            pl.BlockSpec((1, D), lambda i: (0, 0)),
            pl.BlockSpec((1, D), lambda i: (0, 0)),
            pl.BlockSpec((1, D), lambda i: (0, 0)),
        ],
        out_specs=pl.BlockSpec((B, D), lambda i: (i, 0)),
        out_shape=jax.ShapeDtypeStruct((N, D), jnp.float32),
    )(x, p0, p1, W_self, W_neigh, bias, gamma.reshape(1, D),
      beta.reshape(1, D))


def kernel(x, edgeIndex, edgeWeight, W_self, b_self, W_neigh, b_neigh, gamma, beta):
    p0, p1 = _scatter_sc(x, edgeIndex[0], edgeIndex[1], edgeWeight)
    return _post(x, p0, p1, W_self, W_neigh, b_self, b_neigh, gamma, beta)


# speedup vs baseline: 11.6422x; 1.0238x over previous
import functools

import jax
import jax.numpy as jnp
from jax import lax
from jax.experimental import pallas as pl
from jax.experimental.pallas import tpu as pltpu
from jax.experimental.pallas import tpu_sc as plsc

NC = 2
NS = 16
NW = NC * NS
CH = 80


FRAC0 = 0.50


def _scatter_sc(y, src, dst, w):
    N, D = y.shape
    E = src.shape[0]
    n_pt = ((E + 2 * NS * CH - 1) // (2 * NS * CH)) * 2
    n0 = max(4, int(round(FRAC0 * n_pt / 2)) * 2)
    n1 = n_pt - n0
    assert n1 >= 4
    E_pad = NS * n_pt * CH
    pad = E_pad - E
    if pad:
        src = jnp.concatenate([src, jnp.zeros((pad,), jnp.int32)])
        dst = jnp.concatenate([dst, jnp.zeros((pad,), jnp.int32)])
        w = jnp.concatenate([w, jnp.zeros((pad,), jnp.float32)])
    rows_per_tile = ((N + NS * 8 - 1) // (NS * 8)) * 8
    N_pad = rows_per_tile * NS
    zeros_init = jnp.zeros((N_pad, D), jnp.float32)
    nvec = D // 16

    mesh = plsc.VectorSubcoreMesh(core_axis_name="c", subcore_axis_name="s")

    @functools.partial(
        pl.kernel,
        out_type=[
            jax.ShapeDtypeStruct((N_pad, D), jnp.float32),
            jax.ShapeDtypeStruct((N_pad, D), jnp.float32),
        ],
        mesh=mesh,
        scratch_types=[
            pltpu.VMEM_SHARED((N_pad, D), jnp.float32),
            pltpu.VMEM((8, CH), jnp.int32),
            pltpu.VMEM((4, CH), jnp.int32),
            pltpu.VMEM((2, CH), jnp.float32),
            pltpu.VMEM((4, CH, D), jnp.float32),
            pltpu.SemaphoreType.DMA,
            pltpu.SemaphoreType.DMA,
            pltpu.SemaphoreType.DMA,
            pltpu.SemaphoreType.DMA,
            pltpu.SemaphoreType.DMA,
        ],
    )
    def sc_kernel(y_hbm, src_hbm, dst_hbm, w_hbm, zero_hbm, p0_hbm, p1_hbm,
                  acc, src_r, dst_r, w_r, rows,
                  gsem, ssem, srcsem, dstsem, wsem):
        c = lax.axis_index("c")
        s = lax.axis_index("s")
        cbase = jnp.where(c == 0, 0, n0)
        n_ch = jnp.where(c == 0, n0, n1)
        gbase = (s * n_pt + cbase) * CH
        tile_rows = pl.ds(s * rows_per_tile, rows_per_tile)

        def load_src(b):
            pltpu.make_async_copy(
                src_hbm.at[pl.ds(gbase + b * CH, CH)],
                src_r.at[lax.rem(b, 8)], srcsem).start()

        def load_dst(b):
            pltpu.make_async_copy(
                dst_hbm.at[pl.ds(gbase + b * CH, CH)],
                dst_r.at[lax.rem(b, 4)], dstsem).start()

        def load_w(b, slot):
            pltpu.make_async_copy(
                w_hbm.at[pl.ds(gbase + b * CH, CH)], w_r.at[slot],
                wsem).start()

        pltpu.sync_copy(zero_hbm.at[tile_rows], acc.at[tile_rows])

        for b in range(6):
            load_src(b)
        for b in range(4):
            load_dst(b)
        for slot in range(2):
            load_w(slot, slot)
        plsc.subcore_barrier()

        for b in range(3):
            pltpu.make_async_copy(
                src_hbm.at[pl.ds(gbase + b * CH, CH)], src_r.at[b],
                srcsem).wait()
            pltpu.make_async_copy(
                y_hbm.at[src_r.at[b]], rows.at[b], gsem).start()

        n2 = n_ch // 2

        def chunk_pair(b2, carry):
            r2 = lax.rem(b2, 2)
            r4 = lax.rem(b2, 4)
            for slot in range(2):
                b = 2 * b2 + slot
                i4 = 2 * r2 + slot
                i8 = 2 * r4 + slot
                i4p3 = lax.rem(i4 + 3, 4)
                i8p3 = lax.rem(i8 + 3, 8)
                pltpu.make_async_copy(
                    y_hbm.at[src_r.at[i8]], rows.at[i4], gsem).wait()
                pltpu.make_async_copy(
                    w_hbm.at[pl.ds(gbase + b * CH, CH)], w_r.at[slot],
                    wsem).wait()

                for g in range(CH // 16):
                    wv = w_r[slot, pl.ds(g * 16, 16)]
                    for t in range(16):
                        e = g * 16 + t
                        w_e = wv[t]
                        for j in range(nvec):
                            sl = pl.ds(j * 16, 16)
                            rows[i4, e, sl] = rows[i4, e, sl] * w_e

                @pl.when(b + 2 < n_ch)
                def _():
                    load_w(b + 2, slot)

                @pl.when(b >= 1)
                def _():
                    pltpu.make_async_copy(
                        rows.at[i4p3], acc.at[dst_r.at[i4p3]], ssem).wait()

                @pl.when(jnp.logical_and(b >= 1, b + 3 < n_ch))
                def _():
                    load_dst(b + 3)

                @pl.when(b + 6 < n_ch)
                def _():
                    load_src(b + 6)

                @pl.when(b + 3 < n_ch)
                def _():
                    pltpu.make_async_copy(
                        src_hbm.at[pl.ds(gbase + (b + 3) * CH, CH)],
                        src_r.at[i8p3], srcsem).wait()
                    pltpu.make_async_copy(
                        y_hbm.at[src_r.at[i8p3]], rows.at[i4p3], gsem).start()

                pltpu.make_async_copy(
                    dst_hbm.at[pl.ds(gbase + b * CH, CH)], dst_r.at[i4],
                    dstsem).wait()
                pltpu.make_async_copy(
                    rows.at[i4], acc.at[dst_r.at[i4]], ssem
                ).start(add=True)
            return carry

        lax.fori_loop(0, n2, chunk_pair, 0)
        pltpu.make_async_copy(
            rows.at[0], acc.at[dst_r.at[0]], ssem).wait()
        plsc.subcore_barrier()

        @pl.when(c == 0)
        def _():
            pltpu.sync_copy(acc.at[tile_rows], p0_hbm.at[tile_rows])

        @pl.when(c == 1)
        def _():
            pltpu.sync_copy(acc.at[tile_rows], p1_hbm.at[tile_rows])

    return sc_kernel(y, src, dst, w, zeros_init)


def _post_body(x_ref, p0_ref, p1_ref, ws_ref, wn_ref, bias_ref, g_ref, b_ref,
               o_ref):
    dn = (((1,), (1,)), ((), ()))
    h = lax.dot_general(x_ref[...], ws_ref[...], dn,
                        preferred_element_type=jnp.float32)
    h = h + lax.dot_general(p0_ref[...] + p1_ref[...], wn_ref[...], dn,
                            preferred_element_type=jnp.float32)
    h = h + bias_ref[...]
    h = jnp.maximum(h, 0.0)
    mean = jnp.mean(h, axis=1, keepdims=True)
    cen = h - mean
    var = jnp.mean(cen * cen, axis=1, keepdims=True)
    o_ref[...] = cen * lax.rsqrt(var + 1e-5) * g_ref[...] + b_ref[...]


def _post(x, p0, p1, W_self, W_neigh, b_self, b_neigh, gamma, beta):
    N, D = x.shape
    B = 2000 if N % 2000 == 0 else 8
    grid = (N // B,)
    bias = (b_self + b_neigh).reshape(1, D)
    return pl.pallas_call(
        _post_body,
        grid=grid,
        in_specs=[
            pl.BlockSpec((B, D), lambda i: (i, 0)),
            pl.BlockSpec((B, D), lambda i: (i, 0)),
            pl.BlockSpec((B, D), lambda i: (i, 0)),
            pl.BlockSpec((D, D), lambda i: (0, 0)),
            pl.BlockSpec((D, D), lambda i: (0, 0)),
            pl.BlockSpec((1, D), lambda i: (0, 0)),
            pl.BlockSpec((1, D), lambda i: (0, 0)),
            pl.BlockSpec((1, D), lambda i: (0, 0)),
        ],
        out_specs=pl.BlockSpec((B, D), lambda i: (i, 0)),
        out_shape=jax.ShapeDtypeStruct((N, D), jnp.float32),
    )(x, p0, p1, W_self, W_neigh, bias, gamma.reshape(1, D),
      beta.reshape(1, D))


def kernel(x, edgeIndex, edgeWeight, W_self, b_self, W_neigh, b_neigh, gamma, beta):
    p0, p1 = _scatter_sc(x, edgeIndex[0], edgeIndex[1], edgeWeight)
    return _post(x, p0, p1, W_self, W_neigh, b_self, b_neigh, gamma, beta)


# speedup vs baseline: 12.0422x; 1.0344x over previous
import functools

import jax
import jax.numpy as jnp
from jax import lax
from jax.experimental import pallas as pl
from jax.experimental.pallas import tpu as pltpu
from jax.experimental.pallas import tpu_sc as plsc

NC = 2
NS = 16
NW = NC * NS
CH = 80


FRAC0 = 0.50


def _scatter_sc(y, src, dst, w):
    N, D = y.shape
    E = src.shape[0]
    n_pt = ((E + 2 * NS * CH - 1) // (2 * NS * CH)) * 2
    n0 = max(4, int(round(FRAC0 * n_pt / 2)) * 2)
    n1 = n_pt - n0
    assert n1 >= 4
    E_pad = NS * n_pt * CH
    pad = E_pad - E
    if pad:
        src = jnp.concatenate([src, jnp.zeros((pad,), jnp.int32)])
        dst = jnp.concatenate([dst, jnp.zeros((pad,), jnp.int32)])
        w = jnp.concatenate([w, jnp.zeros((pad,), jnp.float32)])
    rows_per_tile = ((N + NS * 8 - 1) // (NS * 8)) * 8
    N_pad = rows_per_tile * NS
    nvec = D // 16

    mesh = plsc.VectorSubcoreMesh(core_axis_name="c", subcore_axis_name="s")

    @functools.partial(
        pl.kernel,
        out_type=[
            jax.ShapeDtypeStruct((N_pad, D), jnp.float32),
            jax.ShapeDtypeStruct((N_pad, D), jnp.float32),
        ],
        mesh=mesh,
        scratch_types=[
            pltpu.VMEM_SHARED((N_pad, D), jnp.float32),
            pltpu.VMEM((8, CH), jnp.int32),
            pltpu.VMEM((4, CH), jnp.int32),
            pltpu.VMEM((2, CH), jnp.float32),
            pltpu.VMEM((4, CH, D), jnp.float32),
            pltpu.VMEM((40, D), jnp.float32),
            pltpu.SemaphoreType.DMA,
            pltpu.SemaphoreType.DMA,
            pltpu.SemaphoreType.DMA,
            pltpu.SemaphoreType.DMA,
            pltpu.SemaphoreType.DMA,
        ],
    )
    def sc_kernel(y_hbm, src_hbm, dst_hbm, w_hbm, p0_hbm, p1_hbm,
                  acc, src_r, dst_r, w_r, rows, zb,
                  gsem, ssem, srcsem, dstsem, wsem):
        c = lax.axis_index("c")
        s = lax.axis_index("s")
        cbase = jnp.where(c == 0, 0, n0)
        n_ch = jnp.where(c == 0, n0, n1)
        gbase = (s * n_pt + cbase) * CH
        tile_rows = pl.ds(s * rows_per_tile, rows_per_tile)

        def load_src(b):
            pltpu.make_async_copy(
                src_hbm.at[pl.ds(gbase + b * CH, CH)],
                src_r.at[lax.rem(b, 8)], srcsem).start()

        def load_dst(b):
            pltpu.make_async_copy(
                dst_hbm.at[pl.ds(gbase + b * CH, CH)],
                dst_r.at[lax.rem(b, 4)], dstsem).start()

        def load_w(b, slot):
            pltpu.make_async_copy(
                w_hbm.at[pl.ds(gbase + b * CH, CH)], w_r.at[slot],
                wsem).start()

        for b in range(6):
            load_src(b)
        for b in range(4):
            load_dst(b)
        for slot in range(2):
            load_w(slot, slot)

        for zi in range(40):
            for zj in range(D // 16):
                zb[zi, pl.ds(zj * 16, 16)] = jnp.zeros((16,), jnp.float32)
        nfull, rem_rows = rows_per_tile // 40, rows_per_tile % 40
        for k in range(nfull):
            pltpu.sync_copy(
                zb, acc.at[pl.ds(s * rows_per_tile + k * 40, 40)])
        if rem_rows:
            pltpu.sync_copy(
                zb.at[pl.ds(0, rem_rows)],
                acc.at[pl.ds(s * rows_per_tile + nfull * 40, rem_rows)])
        plsc.subcore_barrier()

        for b in range(3):
            pltpu.make_async_copy(
                src_hbm.at[pl.ds(gbase + b * CH, CH)], src_r.at[b],
                srcsem).wait()
            pltpu.make_async_copy(
                y_hbm.at[src_r.at[b]], rows.at[b], gsem).start()

        n2 = n_ch // 2

        def chunk_pair(b2, carry):
            r2 = lax.rem(b2, 2)
            r4 = lax.rem(b2, 4)
            for slot in range(2):
                b = 2 * b2 + slot
                i4 = 2 * r2 + slot
                i8 = 2 * r4 + slot
                i4p3 = lax.rem(i4 + 3, 4)
                i8p3 = lax.rem(i8 + 3, 8)
                pltpu.make_async_copy(
                    y_hbm.at[src_r.at[i8]], rows.at[i4], gsem).wait()
                pltpu.make_async_copy(
                    w_hbm.at[pl.ds(gbase + b * CH, CH)], w_r.at[slot],
                    wsem).wait()

                for g in range(CH // 16):
                    wv = w_r[slot, pl.ds(g * 16, 16)]
                    for t in range(16):
                        e = g * 16 + t
                        w_e = wv[t]
                        for j in range(nvec):
                            sl = pl.ds(j * 16, 16)
                            rows[i4, e, sl] = rows[i4, e, sl] * w_e

                @pl.when(b + 2 < n_ch)
                def _():
                    load_w(b + 2, slot)

                @pl.when(b >= 1)
                def _():
                    pltpu.make_async_copy(
                        rows.at[i4p3], acc.at[dst_r.at[i4p3]], ssem).wait()

                @pl.when(jnp.logical_and(b >= 1, b + 3 < n_ch))
                def _():
                    load_dst(b + 3)

                @pl.when(b + 6 < n_ch)
                def _():
                    load_src(b + 6)

                @pl.when(b + 3 < n_ch)
                def _():
                    pltpu.make_async_copy(
                        src_hbm.at[pl.ds(gbase + (b + 3) * CH, CH)],
                        src_r.at[i8p3], srcsem).wait()
                    pltpu.make_async_copy(
                        y_hbm.at[src_r.at[i8p3]], rows.at[i4p3], gsem).start()

                pltpu.make_async_copy(
                    dst_hbm.at[pl.ds(gbase + b * CH, CH)], dst_r.at[i4],
                    dstsem).wait()
                pltpu.make_async_copy(
                    rows.at[i4], acc.at[dst_r.at[i4]], ssem
                ).start(add=True)
            return carry

        lax.fori_loop(0, n2, chunk_pair, 0)
        pltpu.make_async_copy(
            rows.at[0], acc.at[dst_r.at[0]], ssem).wait()
        plsc.subcore_barrier()

        @pl.when(c == 0)
        def _():
            pltpu.sync_copy(acc.at[tile_rows], p0_hbm.at[tile_rows])

        @pl.when(c == 1)
        def _():
            pltpu.sync_copy(acc.at[tile_rows], p1_hbm.at[tile_rows])

    return sc_kernel(y, src, dst, w)


def _post_body(x_ref, p0_ref, p1_ref, ws_ref, wn_ref, bias_ref, g_ref, b_ref,
               o_ref):
    dn = (((1,), (1,)), ((), ()))
    h = lax.dot_general(x_ref[...], ws_ref[...], dn,
                        preferred_element_type=jnp.float32)
    h = h + lax.dot_general(p0_ref[...] + p1_ref[...], wn_ref[...], dn,
                            preferred_element_type=jnp.float32)
    h = h + bias_ref[...]
    h = jnp.maximum(h, 0.0)
    mean = jnp.mean(h, axis=1, keepdims=True)
    cen = h - mean
    var = jnp.mean(cen * cen, axis=1, keepdims=True)
    o_ref[...] = cen * lax.rsqrt(var + 1e-5) * g_ref[...] + b_ref[...]


def _post(x, p0, p1, W_self, W_neigh, b_self, b_neigh, gamma, beta):
    N, D = x.shape
    B = 2000 if N % 2000 == 0 else 8
    grid = (N // B,)
    bias = (b_self + b_neigh).reshape(1, D)
    return pl.pallas_call(
        _post_body,
        grid=grid,
        in_specs=[
            pl.BlockSpec((B, D), lambda i: (i, 0)),
            pl.BlockSpec((B, D), lambda i: (i, 0)),
            pl.BlockSpec((B, D), lambda i: (i, 0)),
            pl.BlockSpec((D, D), lambda i: (0, 0)),
            pl.BlockSpec((D, D), lambda i: (0, 0)),
            pl.BlockSpec((1, D), lambda i: (0, 0)),
            pl.BlockSpec((1, D), lambda i: (0, 0)),
            pl.BlockSpec((1, D), lambda i: (0, 0)),
        ],
        out_specs=pl.BlockSpec((B, D), lambda i: (i, 0)),
        out_shape=jax.ShapeDtypeStruct((N, D), jnp.float32),
    )(x, p0, p1, W_self, W_neigh, bias, gamma.reshape(1, D),
      beta.reshape(1, D))


def kernel(x, edgeIndex, edgeWeight, W_self, b_self, W_neigh, b_neigh, gamma, beta):
    p0, p1 = _scatter_sc(x, edgeIndex[0], edgeIndex[1], edgeWeight)
    return _post(x, p0, p1, W_self, W_neigh, b_self, b_neigh, gamma, beta)


# speedup vs baseline: 12.8635x; 1.0682x over previous
import functools

import jax
import jax.numpy as jnp
from jax import lax
from jax.experimental import pallas as pl
from jax.experimental.pallas import tpu as pltpu
from jax.experimental.pallas import tpu_sc as plsc

NC = 2
NS = 16
NW = NC * NS
CH = 80


FRAC0 = 0.50


def _scatter_sc(y, ei, w):
    N, D = y.shape
    E = ei.shape[1]
    n_pt = ((E + 2 * NS * CH - 1) // (2 * NS * CH)) * 2
    n0 = max(4, int(round(FRAC0 * n_pt / 2)) * 2)
    n1 = n_pt - n0
    assert n1 >= 4
    E_pad = NS * n_pt * CH
    pad = E_pad - E
    if pad:
        ei = jnp.concatenate(
            [ei, jnp.zeros((2, pad), jnp.int32)], axis=1)
        w = jnp.concatenate([w, jnp.zeros((pad,), jnp.float32)])
    er = ei.reshape(2 * E_pad)
    rows_per_tile = ((N + NS * 8 - 1) // (NS * 8)) * 8
    N_pad = rows_per_tile * NS
    nvec = D // 16

    mesh = plsc.VectorSubcoreMesh(core_axis_name="c", subcore_axis_name="s")

    @functools.partial(
        pl.kernel,
        out_type=[
            jax.ShapeDtypeStruct((N_pad, D), jnp.float32),
            jax.ShapeDtypeStruct((N_pad, D), jnp.float32),
        ],
        mesh=mesh,
        scratch_types=[
            pltpu.VMEM_SHARED((N_pad, D), jnp.float32),
            pltpu.VMEM((8, CH), jnp.int32),
            pltpu.VMEM((4, CH), jnp.int32),
            pltpu.VMEM((2, CH), jnp.float32),
            pltpu.VMEM((4, CH, D), jnp.float32),
            pltpu.VMEM((40, D), jnp.float32),
            pltpu.SemaphoreType.DMA,
            pltpu.SemaphoreType.DMA,
            pltpu.SemaphoreType.DMA,
            pltpu.SemaphoreType.DMA,
            pltpu.SemaphoreType.DMA,
        ],
    )
    def sc_kernel(y_hbm, ei_hbm, w_hbm, p0_hbm, p1_hbm,
                  acc, src_r, dst_r, w_r, rows, zb,
                  gsem, ssem, srcsem, dstsem, wsem):
        c = lax.axis_index("c")
        s = lax.axis_index("s")
        cbase = jnp.where(c == 0, 0, n0)
        n_ch = jnp.where(c == 0, n0, n1)
        gbase = (s * n_pt + cbase) * CH
        tile_rows = pl.ds(s * rows_per_tile, rows_per_tile)

        def load_src(b):
            pltpu.make_async_copy(
                ei_hbm.at[pl.ds(gbase + b * CH, CH)],
                src_r.at[lax.rem(b, 8)], srcsem).start()

        def load_dst(b):
            pltpu.make_async_copy(
                ei_hbm.at[pl.ds(E_pad + gbase + b * CH, CH)],
                dst_r.at[lax.rem(b, 4)], dstsem).start()

        def load_w(b, slot):
            pltpu.make_async_copy(
                w_hbm.at[pl.ds(gbase + b * CH, CH)], w_r.at[slot],
                wsem).start()

        for b in range(6):
            load_src(b)
        for b in range(4):
            load_dst(b)
        for slot in range(2):
            load_w(slot, slot)

        for zi in range(40):
            for zj in range(D // 16):
                zb[zi, pl.ds(zj * 16, 16)] = jnp.zeros((16,), jnp.float32)
        nfull, rem_rows = rows_per_tile // 40, rows_per_tile % 40
        for k in range(nfull):
            pltpu.sync_copy(
                zb, acc.at[pl.ds(s * rows_per_tile + k * 40, 40)])
        if rem_rows:
            pltpu.sync_copy(
                zb.at[pl.ds(0, rem_rows)],
                acc.at[pl.ds(s * rows_per_tile + nfull * 40, rem_rows)])
        plsc.subcore_barrier()

        for b in range(3):
            pltpu.make_async_copy(
                ei_hbm.at[pl.ds(gbase + b * CH, CH)], src_r.at[b],
                srcsem).wait()
            pltpu.make_async_copy(
                y_hbm.at[src_r.at[b]], rows.at[b], gsem).start()

        n2 = n_ch // 2

        def chunk_pair(b2, carry):
            r2 = lax.rem(b2, 2)
            r4 = lax.rem(b2, 4)
            for slot in range(2):
                b = 2 * b2 + slot
                i4 = 2 * r2 + slot
                i8 = 2 * r4 + slot
                i4p3 = lax.rem(i4 + 3, 4)
                i8p3 = lax.rem(i8 + 3, 8)
                pltpu.make_async_copy(
                    y_hbm.at[src_r.at[i8]], rows.at[i4], gsem).wait()
                pltpu.make_async_copy(
                    w_hbm.at[pl.ds(gbase + b * CH, CH)], w_r.at[slot],
                    wsem).wait()

                for g in range(CH // 16):
                    wv = w_r[slot, pl.ds(g * 16, 16)]
                    for t in range(16):
                        e = g * 16 + t
                        w_e = wv[t]
                        for j in range(nvec):
                            sl = pl.ds(j * 16, 16)
                            rows[i4, e, sl] = rows[i4, e, sl] * w_e

                @pl.when(b + 2 < n_ch)
                def _():
                    load_w(b + 2, slot)

                @pl.when(b >= 1)
                def _():
                    pltpu.make_async_copy(
                        rows.at[i4p3], acc.at[dst_r.at[i4p3]], ssem).wait()

                @pl.when(jnp.logical_and(b >= 1, b + 3 < n_ch))
                def _():
                    load_dst(b + 3)

                @pl.when(b + 6 < n_ch)
                def _():
                    load_src(b + 6)

                @pl.when(b + 3 < n_ch)
                def _():
                    pltpu.make_async_copy(
                        ei_hbm.at[pl.ds(gbase + (b + 3) * CH, CH)],
                        src_r.at[i8p3], srcsem).wait()
                    pltpu.make_async_copy(
                        y_hbm.at[src_r.at[i8p3]], rows.at[i4p3], gsem).start()

                pltpu.make_async_copy(
                    ei_hbm.at[pl.ds(E_pad + gbase + b * CH, CH)], dst_r.at[i4],
                    dstsem).wait()
                pltpu.make_async_copy(
                    rows.at[i4], acc.at[dst_r.at[i4]], ssem
                ).start(add=True)
            return carry

        lax.fori_loop(0, n2, chunk_pair, 0)
        pltpu.make_async_copy(
            rows.at[0], acc.at[dst_r.at[0]], ssem).wait()
        plsc.subcore_barrier()

        @pl.when(c == 0)
        def _():
            pltpu.sync_copy(acc.at[tile_rows], p0_hbm.at[tile_rows])

        @pl.when(c == 1)
        def _():
            pltpu.sync_copy(acc.at[tile_rows], p1_hbm.at[tile_rows])

    return sc_kernel(y, er, w)


def _post_body(x_ref, p0_ref, p1_ref, ws_ref, wn_ref, bias_ref, g_ref, b_ref,
               o_ref):
    dn = (((1,), (1,)), ((), ()))
    h = lax.dot_general(x_ref[...], ws_ref[...], dn,
                        preferred_element_type=jnp.float32)
    h = h + lax.dot_general(p0_ref[...] + p1_ref[...], wn_ref[...], dn,
                            preferred_element_type=jnp.float32)
    h = h + bias_ref[...]
    h = jnp.maximum(h, 0.0)
    mean = jnp.mean(h, axis=1, keepdims=True)
    cen = h - mean
    var = jnp.mean(cen * cen, axis=1, keepdims=True)
    o_ref[...] = cen * lax.rsqrt(var + 1e-5) * g_ref[...] + b_ref[...]


def _post(x, p0, p1, W_self, W_neigh, b_self, b_neigh, gamma, beta):
    N, D = x.shape
    B = 2000 if N % 2000 == 0 else 8
    grid = (N // B,)
    bias = (b_self + b_neigh).reshape(1, D)
    return pl.pallas_call(
        _post_body,
        grid=grid,
        in_specs=[
            pl.BlockSpec((B, D), lambda i: (i, 0)),
            pl.BlockSpec((B, D), lambda i: (i, 0)),
            pl.BlockSpec((B, D), lambda i: (i, 0)),
            pl.BlockSpec((D, D), lambda i: (0, 0)),
            pl.BlockSpec((D, D), lambda i: (0, 0)),
            pl.BlockSpec((1, D), lambda i: (0, 0)),
            pl.BlockSpec((1, D), lambda i: (0, 0)),
            pl.BlockSpec((1, D), lambda i: (0, 0)),
        ],
        out_specs=pl.BlockSpec((B, D), lambda i: (i, 0)),
        out_shape=jax.ShapeDtypeStruct((N, D), jnp.float32),
    )(x, p0, p1, W_self, W_neigh, bias, gamma.reshape(1, D),
      beta.reshape(1, D))


def kernel(x, edgeIndex, edgeWeight, W_self, b_self, W_neigh, b_neigh, gamma, beta):
    p0, p1 = _scatter_sc(x, edgeIndex, edgeWeight)
    return _post(x, p0, p1, W_self, W_neigh, b_self, b_neigh, gamma, beta)
